# Initial kernel scaffold; baseline (speedup 1.0000x reference)
import jax
import jax.numpy as jnp
from jax.experimental import pallas as pl


def kernel(x, edge_index, batch, W1, b1, W2, b2, W3, b3, W4, b4, bn1_g, bn1_b, bn1_m, bn1_v, bn2_g, bn2_b, bn2_m, bn2_v):
    raise NotImplementedError("write your pallas kernel here")


# speedup vs baseline: 24.1002x; 24.1002x over previous
import functools

import jax
import jax.numpy as jnp
from jax import lax
from jax.experimental import pallas as pl
from jax.experimental.pallas import tpu as pltpu
from jax.experimental.pallas import tpu_sc as plsc

N_NODES = 100000
N_EDGES = 3200000
N_GRAPHS = 128

CHUNK = 125
EDGE_ROWS = N_EDGES // CHUNK
N_WORKERS = 32
ROWS_PER_TILE = EDGE_ROWS // N_WORKERS
KB = 16
N_BLOCKS = ROWS_PER_TILE // KB
INIT_STEP = 6248
INIT_TAIL = N_NODES - 16 * INIT_STEP

TC_BLOCK = 1000
TC_GRID = N_NODES // TC_BLOCK


@functools.cache
def _sc_agg(feat):
  mesh = plsc.VectorSubcoreMesh(core_axis_name="c", subcore_axis_name="s",
                                num_cores=2, num_subcores=16)

  @functools.partial(
      pl.kernel,
      mesh=mesh,
      out_type=jax.ShapeDtypeStruct((2, N_NODES, feat), jnp.float32),
      scratch_types=[
          pltpu.VMEM((KB, CHUNK), jnp.int32),
          pltpu.VMEM((KB, CHUNK), jnp.int32),
          pltpu.VMEM((2, CHUNK, feat), jnp.float32),
          pltpu.VMEM_SHARED((N_NODES, feat), jnp.float32),
          pltpu.SemaphoreType.DMA,
          pltpu.SemaphoreType.DMA,
      ],
      compiler_params=pltpu.CompilerParams(use_tc_tiling_on_sc=False),
  )
  def sc_agg(src_hbm, dst_hbm, table_hbm, init_hbm, out_hbm,
             src_v, dst_v, rows_v, acc_sh, sem0, sem1):
    c = lax.axis_index("c")
    s = lax.axis_index("s")
    sems = (sem0, sem1)

    ibase = s * INIT_STEP
    pltpu.sync_copy(init_hbm.at[c, pl.ds(ibase, INIT_STEP)],
                    acc_sh.at[pl.ds(ibase, INIT_STEP)])

    @pl.when(s == 15)
    def _():
      pltpu.sync_copy(init_hbm.at[c, pl.ds(16 * INIT_STEP, INIT_TAIL)],
                      acc_sh.at[pl.ds(16 * INIT_STEP, INIT_TAIL)])

    plsc.subcore_barrier()

    wid = c * 16 + s
    row0 = wid * ROWS_PER_TILE

    def block(b, carry):
      rbase = row0 + b * KB
      pltpu.sync_copy(src_hbm.at[pl.ds(rbase, KB)], src_v)
      pltpu.sync_copy(dst_hbm.at[pl.ds(rbase, KB)], dst_v)

      def gstart(k):
        return pltpu.async_copy(table_hbm.at[src_v.at[k]],
                                rows_v.at[k % 2], sems[k % 2])

      descs = {0: gstart(0)}
      for k in range(KB):
        if k + 1 < KB:
          descs[k + 1] = gstart(k + 1)
        descs[k].wait()
        pltpu.sync_copy(rows_v.at[k % 2], acc_sh.at[dst_v.at[k]], add=True)
      return carry

    lax.fori_loop(0, N_BLOCKS, block, 0)
    plsc.subcore_barrier()

    pltpu.sync_copy(acc_sh.at[pl.ds(ibase, INIT_STEP)],
                    out_hbm.at[c, pl.ds(ibase, INIT_STEP)])

    @pl.when(s == 15)
    def _():
      pltpu.sync_copy(acc_sh.at[pl.ds(16 * INIT_STEP, INIT_TAIL)],
                      out_hbm.at[c, pl.ds(16 * INIT_STEP, INIT_TAIL)])

  return sc_agg


def _tc1_body(x_ref, parts_ref, w1_ref, b1_ref, w2_ref, b2_ref, w3_ref,
              b3_ref, g1_ref, bb1_ref, m1_ref, v1_ref, ptab_ref, init2_ref):
  h0 = x_ref[...] + parts_ref[0] + parts_ref[1]
  h1 = jnp.maximum(jnp.dot(h0, w1_ref[...],
                           preferred_element_type=jnp.float32) + b1_ref[...],
                   0.0)
  h = jnp.dot(h1, w2_ref[...], preferred_element_type=jnp.float32) + b2_ref[...]
  scale = g1_ref[...] * lax.rsqrt(v1_ref[...] + 1e-5)
  h = (h - m1_ref[...]) * scale + bb1_ref[...]
  h = jnp.maximum(h, 0.0)
  p = jnp.dot(h, w3_ref[...], preferred_element_type=jnp.float32)
  ptab_ref[...] = p
  init2_ref[0] = p + b3_ref[...]
  init2_ref[1] = jnp.zeros_like(p)


def _tc2_body(parts_ref, batch_ref, w4_ref, b4_ref, g2_ref, bb2_ref, m2_ref,
              v2_ref, out_ref, sums_acc, cnts_acc):
  i = pl.program_id(0)

  @pl.when(i == 0)
  def _():
    sums_acc[...] = jnp.zeros_like(sums_acc)
    cnts_acc[...] = jnp.zeros_like(cnts_acc)

  h2p = parts_ref[0] + parts_ref[1]
  r = jnp.maximum(h2p, 0.0)
  h2 = jnp.dot(r, w4_ref[...], preferred_element_type=jnp.float32) + b4_ref[...]
  scale = g2_ref[...] * lax.rsqrt(v2_ref[...] + 1e-5)
  h2 = (h2 - m2_ref[...]) * scale + bb2_ref[...]
  h2 = jnp.maximum(h2, 0.0)

  seg = batch_ref[0]
  gids = lax.broadcasted_iota(jnp.int32, (N_GRAPHS, TC_BLOCK), 0)
  onehot = (gids == seg).astype(jnp.float32)
  sums_acc[...] += jnp.dot(onehot, h2, preferred_element_type=jnp.float32)
  cnts_acc[...] += jnp.sum(onehot, axis=1, keepdims=True)

  @pl.when(i == TC_GRID - 1)
  def _():
    out_ref[...] = sums_acc[...] / jnp.maximum(cnts_acc[...], 1.0)


def kernel(x, edge_index, batch, W1, b1, W2, b2, W3, b3, W4, b4,
           bn1_g, bn1_b, bn1_m, bn1_v, bn2_g, bn2_b, bn2_m, bn2_v):
  src2d = edge_index[0].reshape(EDGE_ROWS, CHUNK).astype(jnp.int32)
  dst2d = edge_index[1].reshape(EDGE_ROWS, CHUNK).astype(jnp.int32)
  init1 = jnp.zeros((2, N_NODES, 16), jnp.float32)

  parts1 = _sc_agg(16)(src2d, dst2d, x, init1)

  row = lambda a: a.reshape(1, -1)
  full = lambda shape: pl.BlockSpec(shape, lambda i: (0,) * len(shape))

  ptab, init2 = pl.pallas_call(
      _tc1_body,
      grid=(TC_GRID,),
      in_specs=[
          pl.BlockSpec((TC_BLOCK, 16), lambda i: (i, 0)),
          pl.BlockSpec((2, TC_BLOCK, 16), lambda i: (0, i, 0)),
          full((16, 64)), full((1, 64)), full((64, 64)), full((1, 64)),
          full((64, 8)), full((1, 8)),
          full((1, 64)), full((1, 64)), full((1, 64)), full((1, 64)),
      ],
      out_specs=[
          pl.BlockSpec((TC_BLOCK, 8), lambda i: (i, 0)),
          pl.BlockSpec((2, TC_BLOCK, 8), lambda i: (0, i, 0)),
      ],
      out_shape=[
          jax.ShapeDtypeStruct((N_NODES, 8), jnp.float32),
          jax.ShapeDtypeStruct((2, N_NODES, 8), jnp.float32),
      ],
  )(x, parts1, W1, row(b1), W2, row(b2), W3, row(b3),
    row(bn1_g), row(bn1_b), row(bn1_m), row(bn1_v))

  parts2 = _sc_agg(8)(src2d, dst2d, ptab, init2)

  batch3d = batch.astype(jnp.int32).reshape(TC_GRID, 1, TC_BLOCK)
  out = pl.pallas_call(
      _tc2_body,
      grid=(TC_GRID,),
      in_specs=[
          pl.BlockSpec((2, TC_BLOCK, 8), lambda i: (0, i, 0)),
          pl.BlockSpec((1, 1, TC_BLOCK), lambda i: (i, 0, 0)),
          full((8, 8)), full((1, 8)),
          full((1, 8)), full((1, 8)), full((1, 8)), full((1, 8)),
      ],
      out_specs=pl.BlockSpec((N_GRAPHS, 8), lambda i: (0, 0)),
      out_shape=jax.ShapeDtypeStruct((N_GRAPHS, 8), jnp.float32),
      scratch_shapes=[
          pltpu.VMEM((N_GRAPHS, 8), jnp.float32),
          pltpu.VMEM((N_GRAPHS, 1), jnp.float32),
      ],
  )(parts2, batch3d, W4, row(b4),
    row(bn2_g), row(bn2_b), row(bn2_m), row(bn2_v))

  return out[:, None, :]


# speedup vs baseline: 34.7143x; 1.4404x over previous
import functools

import jax
import jax.numpy as jnp
from jax import lax
from jax.experimental import pallas as pl
from jax.experimental.pallas import tpu as pltpu
from jax.experimental.pallas import tpu_sc as plsc

N_NODES = 100000
N_EDGES = 3200000
N_GRAPHS = 128

CHUNK = 125
EDGE_ROWS = N_EDGES // CHUNK
N_WORKERS = 32
ROWS_PER_TILE = EDGE_ROWS // N_WORKERS
KB = 16
N_BLOCKS = ROWS_PER_TILE // KB
NSLOT = 8
INFLIGHT = 4
INIT_STEP = 6248
INIT_TAIL = N_NODES - 16 * INIT_STEP
ZROWS = INIT_STEP + INIT_TAIL

TC_BLOCK = 1000
TC_GRID = N_NODES // TC_BLOCK


@functools.cache
def _sc_agg(feat):
  mesh = plsc.VectorSubcoreMesh(core_axis_name="c", subcore_axis_name="s",
                                num_cores=2, num_subcores=16)

  @functools.partial(
      pl.kernel,
      mesh=mesh,
      out_type=jax.ShapeDtypeStruct((2, N_NODES, feat), jnp.float32),
      scratch_types=[
          pltpu.VMEM((KB, CHUNK), jnp.int32),
          pltpu.VMEM((KB, CHUNK), jnp.int32),
          pltpu.VMEM((NSLOT, CHUNK, feat), jnp.float32),
          pltpu.VMEM_SHARED((N_NODES, feat), jnp.float32),
          pltpu.SemaphoreType.DMA((NSLOT,)),
          pltpu.SemaphoreType.DMA((NSLOT,)),
      ],
      compiler_params=pltpu.CompilerParams(use_tc_tiling_on_sc=False),
  )
  def sc_agg(src_hbm, dst_hbm, table_hbm, zeros_hbm, out_hbm,
             src_v, dst_v, rows_v, acc_sh, gsem, ssem):
    c = lax.axis_index("c")
    s = lax.axis_index("s")

    ibase = s * INIT_STEP

    @pl.when(c == 0)
    def _():
      pltpu.sync_copy(table_hbm.at[pl.ds(ibase, INIT_STEP)],
                      acc_sh.at[pl.ds(ibase, INIT_STEP)])

      @pl.when(s == 15)
      def _():
        pltpu.sync_copy(table_hbm.at[pl.ds(16 * INIT_STEP, INIT_TAIL)],
                        acc_sh.at[pl.ds(16 * INIT_STEP, INIT_TAIL)])

    @pl.when(c == 1)
    def _():
      pltpu.sync_copy(zeros_hbm.at[pl.ds(0, INIT_STEP)],
                      acc_sh.at[pl.ds(ibase, INIT_STEP)])

      @pl.when(s == 15)
      def _():
        pltpu.sync_copy(zeros_hbm.at[pl.ds(0, INIT_TAIL)],
                        acc_sh.at[pl.ds(16 * INIT_STEP, INIT_TAIL)])

    plsc.subcore_barrier()

    wid = c * 16 + s
    row0 = wid * ROWS_PER_TILE

    def block(b, carry):
      rbase = row0 + b * KB
      pltpu.sync_copy(src_hbm.at[pl.ds(rbase, KB)], src_v)
      pltpu.sync_copy(dst_hbm.at[pl.ds(rbase, KB)], dst_v)

      def gstart(k):
        return pltpu.async_copy(table_hbm.at[src_v.at[k]],
                                rows_v.at[k % NSLOT], gsem.at[k % NSLOT])

      def sstart(k):
        return pltpu.async_copy(rows_v.at[k % NSLOT],
                                acc_sh.at[dst_v.at[k]], ssem.at[k % NSLOT],
                                add=True)

      gd = {}
      sd = {}
      for k in range(INFLIGHT):
        gd[k] = gstart(k)
      for k in range(KB):
        nk = k + INFLIGHT
        if nk < KB:
          if nk >= NSLOT:
            sd[nk - NSLOT].wait()
          gd[nk] = gstart(nk)
        gd[k].wait()
        sd[k] = sstart(k)
      for k in range(KB - NSLOT, KB):
        sd[k].wait()
      return carry

    lax.fori_loop(0, N_BLOCKS, block, 0)
    plsc.subcore_barrier()

    pltpu.sync_copy(acc_sh.at[pl.ds(ibase, INIT_STEP)],
                    out_hbm.at[c, pl.ds(ibase, INIT_STEP)])

    @pl.when(s == 15)
    def _():
      pltpu.sync_copy(acc_sh.at[pl.ds(16 * INIT_STEP, INIT_TAIL)],
                      out_hbm.at[c, pl.ds(16 * INIT_STEP, INIT_TAIL)])

  return sc_agg


def _tc1_body(parts_ref, w1_ref, b1_ref, w2_ref, b2_ref, w3_ref,
              g1_ref, bb1_ref, m1_ref, v1_ref, ptab_ref):
  h0 = parts_ref[0] + parts_ref[1]
  h1 = jnp.maximum(jnp.dot(h0, w1_ref[...],
                           preferred_element_type=jnp.float32) + b1_ref[...],
                   0.0)
  h = jnp.dot(h1, w2_ref[...], preferred_element_type=jnp.float32) + b2_ref[...]
  scale = g1_ref[...] * lax.rsqrt(v1_ref[...] + 1e-5)
  h = (h - m1_ref[...]) * scale + bb1_ref[...]
  h = jnp.maximum(h, 0.0)
  ptab_ref[...] = jnp.dot(h, w3_ref[...], preferred_element_type=jnp.float32)


def _tc2_body(parts_ref, batch_ref, b3_ref, w4_ref, b4_ref, g2_ref, bb2_ref,
              m2_ref, v2_ref, out_ref, sums_acc, cnts_acc):
  i = pl.program_id(0)

  @pl.when(i == 0)
  def _():
    sums_acc[...] = jnp.zeros_like(sums_acc)
    cnts_acc[...] = jnp.zeros_like(cnts_acc)

  h2p = parts_ref[0] + parts_ref[1] + b3_ref[...]
  r = jnp.maximum(h2p, 0.0)
  h2 = jnp.dot(r, w4_ref[...], preferred_element_type=jnp.float32) + b4_ref[...]
  scale = g2_ref[...] * lax.rsqrt(v2_ref[...] + 1e-5)
  h2 = (h2 - m2_ref[...]) * scale + bb2_ref[...]
  h2 = jnp.maximum(h2, 0.0)

  seg = batch_ref[0]
  gids = lax.broadcasted_iota(jnp.int32, (N_GRAPHS, TC_BLOCK), 0)
  onehot = (gids == seg).astype(jnp.float32)
  sums_acc[...] += jnp.dot(onehot, h2, preferred_element_type=jnp.float32)
  cnts_acc[...] += jnp.sum(onehot, axis=1, keepdims=True)

  @pl.when(i == TC_GRID - 1)
  def _():
    out_ref[...] = sums_acc[...] / jnp.maximum(cnts_acc[...], 1.0)


def kernel(x, edge_index, batch, W1, b1, W2, b2, W3, b3, W4, b4,
           bn1_g, bn1_b, bn1_m, bn1_v, bn2_g, bn2_b, bn2_m, bn2_v):
  src2d = edge_index[0].reshape(EDGE_ROWS, CHUNK).astype(jnp.int32)
  dst2d = edge_index[1].reshape(EDGE_ROWS, CHUNK).astype(jnp.int32)
  zeros16 = jnp.zeros((ZROWS, 16), jnp.float32)
  zeros8 = jnp.zeros((ZROWS, 8), jnp.float32)

  parts1 = _sc_agg(16)(src2d, dst2d, x, zeros16)

  row = lambda a: a.reshape(1, -1)
  full = lambda shape: pl.BlockSpec(shape, lambda i: (0,) * len(shape))

  ptab = pl.pallas_call(
      _tc1_body,
      grid=(TC_GRID,),
      in_specs=[
          pl.BlockSpec((2, TC_BLOCK, 16), lambda i: (0, i, 0)),
          full((16, 64)), full((1, 64)), full((64, 64)), full((1, 64)),
          full((64, 8)),
          full((1, 64)), full((1, 64)), full((1, 64)), full((1, 64)),
      ],
      out_specs=pl.BlockSpec((TC_BLOCK, 8), lambda i: (i, 0)),
      out_shape=jax.ShapeDtypeStruct((N_NODES, 8), jnp.float32),
  )(parts1, W1, row(b1), W2, row(b2), W3,
    row(bn1_g), row(bn1_b), row(bn1_m), row(bn1_v))

  parts2 = _sc_agg(8)(src2d, dst2d, ptab, zeros8)

  batch3d = batch.astype(jnp.int32).reshape(TC_GRID, 1, TC_BLOCK)
  out = pl.pallas_call(
      _tc2_body,
      grid=(TC_GRID,),
      in_specs=[
          pl.BlockSpec((2, TC_BLOCK, 8), lambda i: (0, i, 0)),
          pl.BlockSpec((1, 1, TC_BLOCK), lambda i: (i, 0, 0)),
          full((1, 8)), full((8, 8)), full((1, 8)),
          full((1, 8)), full((1, 8)), full((1, 8)), full((1, 8)),
      ],
      out_specs=pl.BlockSpec((N_GRAPHS, 8), lambda i: (0, 0)),
      out_shape=jax.ShapeDtypeStruct((N_GRAPHS, 8), jnp.float32),
      scratch_shapes=[
          pltpu.VMEM((N_GRAPHS, 8), jnp.float32),
          pltpu.VMEM((N_GRAPHS, 1), jnp.float32),
      ],
  )(parts2, batch3d, row(b3), W4, row(b4),
    row(bn2_g), row(bn2_b), row(bn2_m), row(bn2_v))

  return out[:, None, :]


# speedup vs baseline: 52.1510x; 1.5023x over previous
import functools

import jax
import jax.numpy as jnp
from jax import lax
from jax.experimental import pallas as pl
from jax.experimental.pallas import tpu as pltpu
from jax.experimental.pallas import tpu_sc as plsc

N_NODES = 100000
N_EDGES = 3200000
N_GRAPHS = 128

N_WORKERS = 32
EDGES_PER_TILE = N_EDGES // N_WORKERS
CHUNK_BY_FEAT = {16: 400, 8: 1000}
NSLOT = 4
INIT_STEP = 6248
INIT_TAIL = N_NODES - 16 * INIT_STEP
ZROWS = INIT_STEP + INIT_TAIL

TC_BLOCK = 1000
TC_GRID = N_NODES // TC_BLOCK


@functools.cache
def _sc_agg(feat):
  mesh = plsc.VectorSubcoreMesh(core_axis_name="c", subcore_axis_name="s",
                                num_cores=2, num_subcores=16)
  CHUNK = CHUNK_BY_FEAT[feat]
  NCHUNK = EDGES_PER_TILE // CHUNK

  @functools.partial(
      pl.kernel,
      mesh=mesh,
      out_type=jax.ShapeDtypeStruct((2, N_NODES, feat), jnp.float32),
      scratch_types=[
          pltpu.VMEM((NSLOT, CHUNK), jnp.int32),
          pltpu.VMEM((NSLOT, CHUNK), jnp.int32),
          pltpu.VMEM((NSLOT, CHUNK, feat), jnp.float32),
          pltpu.VMEM_SHARED((N_NODES, feat), jnp.float32),
          pltpu.SemaphoreType.DMA((NSLOT,)),
          pltpu.SemaphoreType.DMA((NSLOT,)),
          pltpu.SemaphoreType.DMA((NSLOT,)),
      ],
      compiler_params=pltpu.CompilerParams(use_tc_tiling_on_sc=False),
  )
  def sc_agg(edge_hbm, table_hbm, zeros_hbm, out_hbm,
             src_v, dst_v, rows_v, acc_sh, gsem, ssem, isem):
    c = lax.axis_index("c")
    s = lax.axis_index("s")

    ibase = s * INIT_STEP

    @pl.when(c == 0)
    def _():
      pltpu.sync_copy(table_hbm.at[pl.ds(ibase, INIT_STEP)],
                      acc_sh.at[pl.ds(ibase, INIT_STEP)])

      @pl.when(s == 15)
      def _():
        pltpu.sync_copy(table_hbm.at[pl.ds(16 * INIT_STEP, INIT_TAIL)],
                        acc_sh.at[pl.ds(16 * INIT_STEP, INIT_TAIL)])

    @pl.when(c == 1)
    def _():
      pltpu.sync_copy(zeros_hbm.at[pl.ds(0, INIT_STEP)],
                      acc_sh.at[pl.ds(ibase, INIT_STEP)])

      @pl.when(s == 15)
      def _():
        pltpu.sync_copy(zeros_hbm.at[pl.ds(0, INIT_TAIL)],
                        acc_sh.at[pl.ds(16 * INIT_STEP, INIT_TAIL)])

    plsc.subcore_barrier()

    wid = c * 16 + s
    ebase0 = wid * EDGES_PER_TILE

    def istart(j, jm):
      eb = ebase0 + j * CHUNK
      pltpu.async_copy(edge_hbm.at[0, pl.ds(eb, CHUNK)], src_v.at[jm],
                       isem.at[jm])
      pltpu.async_copy(edge_hbm.at[1, pl.ds(eb, CHUNK)], dst_v.at[jm],
                       isem.at[jm])

    def iwait(jm):
      pltpu.make_async_copy(edge_hbm.at[0, pl.ds(0, CHUNK)], src_v.at[jm],
                            isem.at[jm]).wait()
      pltpu.make_async_copy(edge_hbm.at[1, pl.ds(0, CHUNK)], dst_v.at[jm],
                            isem.at[jm]).wait()

    def gstart(jm):
      pltpu.async_copy(table_hbm.at[src_v.at[jm]], rows_v.at[jm],
                       gsem.at[jm])

    def gwait(jm):
      pltpu.make_async_copy(table_hbm.at[pl.ds(0, CHUNK)], rows_v.at[jm],
                            gsem.at[jm]).wait()

    def sstart(jm):
      pltpu.async_copy(rows_v.at[jm], acc_sh.at[dst_v.at[jm]], ssem.at[jm],
                       add=True)

    def swait(jm):
      pltpu.make_async_copy(rows_v.at[jm], acc_sh.at[pl.ds(0, CHUNK)],
                            ssem.at[jm]).wait()

    istart(0, 0)
    istart(1, 1)
    iwait(0)
    gstart(0)

    def step(k, carry):
      km = k % NSLOT
      k1m = (k + 1) % NSLOT
      k2m = (k + 2) % NSLOT

      @pl.when(k >= 2)
      def _():
        swait(k2m)

      @pl.when(k + 2 < NCHUNK)
      def _():
        istart(k + 2, k2m)

      @pl.when(k + 1 < NCHUNK)
      def _():
        iwait(k1m)
        gstart(k1m)

      gwait(km)
      sstart(km)
      return carry

    lax.fori_loop(0, NCHUNK, step, 0)
    swait((NCHUNK - 2) % NSLOT)
    swait((NCHUNK - 1) % NSLOT)
    plsc.subcore_barrier()

    pltpu.sync_copy(acc_sh.at[pl.ds(ibase, INIT_STEP)],
                    out_hbm.at[c, pl.ds(ibase, INIT_STEP)])

    @pl.when(s == 15)
    def _():
      pltpu.sync_copy(acc_sh.at[pl.ds(16 * INIT_STEP, INIT_TAIL)],
                      out_hbm.at[c, pl.ds(16 * INIT_STEP, INIT_TAIL)])

  return sc_agg


def _tc1_body(parts_ref, w1_ref, b1_ref, w2_ref, b2_ref, w3_ref,
              g1_ref, bb1_ref, m1_ref, v1_ref, ptab_ref):
  h0 = parts_ref[0] + parts_ref[1]
  h1 = jnp.maximum(jnp.dot(h0, w1_ref[...],
                           preferred_element_type=jnp.float32) + b1_ref[...],
                   0.0)
  h = jnp.dot(h1, w2_ref[...], preferred_element_type=jnp.float32) + b2_ref[...]
  scale = g1_ref[...] * lax.rsqrt(v1_ref[...] + 1e-5)
  h = (h - m1_ref[...]) * scale + bb1_ref[...]
  h = jnp.maximum(h, 0.0)
  ptab_ref[...] = jnp.dot(h, w3_ref[...], preferred_element_type=jnp.float32)


def _tc2_body(parts_ref, batch_ref, b3_ref, w4_ref, b4_ref, g2_ref, bb2_ref,
              m2_ref, v2_ref, out_ref, sums_acc, cnts_acc):
  i = pl.program_id(0)

  @pl.when(i == 0)
  def _():
    sums_acc[...] = jnp.zeros_like(sums_acc)
    cnts_acc[...] = jnp.zeros_like(cnts_acc)

  h2p = parts_ref[0] + parts_ref[1] + b3_ref[...]
  r = jnp.maximum(h2p, 0.0)
  h2 = jnp.dot(r, w4_ref[...], preferred_element_type=jnp.float32) + b4_ref[...]
  scale = g2_ref[...] * lax.rsqrt(v2_ref[...] + 1e-5)
  h2 = (h2 - m2_ref[...]) * scale + bb2_ref[...]
  h2 = jnp.maximum(h2, 0.0)

  seg = batch_ref[0]
  gids = lax.broadcasted_iota(jnp.int32, (N_GRAPHS, TC_BLOCK), 0)
  onehot = (gids == seg).astype(jnp.float32)
  sums_acc[...] += jnp.dot(onehot, h2, preferred_element_type=jnp.float32)
  cnts_acc[...] += jnp.sum(onehot, axis=1, keepdims=True)

  @pl.when(i == TC_GRID - 1)
  def _():
    out_ref[...] = sums_acc[...] / jnp.maximum(cnts_acc[...], 1.0)


def kernel(x, edge_index, batch, W1, b1, W2, b2, W3, b3, W4, b4,
           bn1_g, bn1_b, bn1_m, bn1_v, bn2_g, bn2_b, bn2_m, bn2_v):
  zeros16 = jnp.zeros((ZROWS, 16), jnp.float32)
  zeros8 = jnp.zeros((ZROWS, 8), jnp.float32)

  parts1 = _sc_agg(16)(edge_index, x, zeros16)

  row = lambda a: a.reshape(1, -1)
  full = lambda shape: pl.BlockSpec(shape, lambda i: (0,) * len(shape))

  ptab = pl.pallas_call(
      _tc1_body,
      grid=(TC_GRID,),
      in_specs=[
          pl.BlockSpec((2, TC_BLOCK, 16), lambda i: (0, i, 0)),
          full((16, 64)), full((1, 64)), full((64, 64)), full((1, 64)),
          full((64, 8)),
          full((1, 64)), full((1, 64)), full((1, 64)), full((1, 64)),
      ],
      out_specs=pl.BlockSpec((TC_BLOCK, 8), lambda i: (i, 0)),
      out_shape=jax.ShapeDtypeStruct((N_NODES, 8), jnp.float32),
  )(parts1, W1, row(b1), W2, row(b2), W3,
    row(bn1_g), row(bn1_b), row(bn1_m), row(bn1_v))

  parts2 = _sc_agg(8)(edge_index, ptab, zeros8)

  batch3d = batch.astype(jnp.int32).reshape(TC_GRID, 1, TC_BLOCK)
  out = pl.pallas_call(
      _tc2_body,
      grid=(TC_GRID,),
      in_specs=[
          pl.BlockSpec((2, TC_BLOCK, 8), lambda i: (0, i, 0)),
          pl.BlockSpec((1, 1, TC_BLOCK), lambda i: (i, 0, 0)),
          full((1, 8)), full((8, 8)), full((1, 8)),
          full((1, 8)), full((1, 8)), full((1, 8)), full((1, 8)),
      ],
      out_specs=pl.BlockSpec((N_GRAPHS, 8), lambda i: (0, 0)),
      out_shape=jax.ShapeDtypeStruct((N_GRAPHS, 8), jnp.float32),
      scratch_shapes=[
          pltpu.VMEM((N_GRAPHS, 8), jnp.float32),
          pltpu.VMEM((N_GRAPHS, 1), jnp.float32),
      ],
  )(parts2, batch3d, row(b3), W4, row(b4),
    row(bn2_g), row(bn2_b), row(bn2_m), row(bn2_v))

  return out[:, None, :]


# speedup vs baseline: 60.2208x; 1.1547x over previous
import functools

import jax
import jax.numpy as jnp
from jax import lax
from jax.experimental import pallas as pl
from jax.experimental.pallas import tpu as pltpu
from jax.experimental.pallas import tpu_sc as plsc

N_NODES = 100000
N_EDGES = 3200000
N_GRAPHS = 128

N_WORKERS = 32
EDGES_PER_TILE = N_EDGES // N_WORKERS
CHUNK_BY_FEAT = {16: 400, 8: 1000}
NSLOT_BY_FEAT = {16: 4, 8: 6}
INIT_STEP = 6248
INIT_TAIL = N_NODES - 16 * INIT_STEP
ZROWS = INIT_STEP + INIT_TAIL

TC_BLOCK = 4000
TC_GRID = N_NODES // TC_BLOCK


@functools.cache
def _sc_agg(feat):
  mesh = plsc.VectorSubcoreMesh(core_axis_name="c", subcore_axis_name="s",
                                num_cores=2, num_subcores=16)
  CHUNK = CHUNK_BY_FEAT[feat]
  NCHUNK = EDGES_PER_TILE // CHUNK
  NSLOT = NSLOT_BY_FEAT[feat]
  SWAIT_LAG = NSLOT - 2

  @functools.partial(
      pl.kernel,
      mesh=mesh,
      out_type=jax.ShapeDtypeStruct((2, N_NODES, feat), jnp.float32),
      scratch_types=[
          pltpu.VMEM((NSLOT, CHUNK), jnp.int32),
          pltpu.VMEM((NSLOT, CHUNK), jnp.int32),
          pltpu.VMEM((NSLOT, CHUNK, feat), jnp.float32),
          pltpu.VMEM_SHARED((N_NODES, feat), jnp.float32),
          pltpu.SemaphoreType.DMA((NSLOT,)),
          pltpu.SemaphoreType.DMA((NSLOT,)),
          pltpu.SemaphoreType.DMA((NSLOT,)),
      ],
      compiler_params=pltpu.CompilerParams(use_tc_tiling_on_sc=False),
  )
  def sc_agg(edge_hbm, table_hbm, zeros_hbm, out_hbm,
             src_v, dst_v, rows_v, acc_sh, gsem, ssem, isem):
    c = lax.axis_index("c")
    s = lax.axis_index("s")

    ibase = s * INIT_STEP

    @pl.when(c == 0)
    def _():
      pltpu.sync_copy(table_hbm.at[pl.ds(ibase, INIT_STEP)],
                      acc_sh.at[pl.ds(ibase, INIT_STEP)])

      @pl.when(s == 15)
      def _():
        pltpu.sync_copy(table_hbm.at[pl.ds(16 * INIT_STEP, INIT_TAIL)],
                        acc_sh.at[pl.ds(16 * INIT_STEP, INIT_TAIL)])

    @pl.when(c == 1)
    def _():
      pltpu.sync_copy(zeros_hbm.at[pl.ds(0, INIT_STEP)],
                      acc_sh.at[pl.ds(ibase, INIT_STEP)])

      @pl.when(s == 15)
      def _():
        pltpu.sync_copy(zeros_hbm.at[pl.ds(0, INIT_TAIL)],
                        acc_sh.at[pl.ds(16 * INIT_STEP, INIT_TAIL)])

    plsc.subcore_barrier()

    wid = c * 16 + s
    ebase0 = wid * EDGES_PER_TILE

    def istart(j, jm):
      eb = ebase0 + j * CHUNK
      pltpu.async_copy(edge_hbm.at[0, pl.ds(eb, CHUNK)], src_v.at[jm],
                       isem.at[jm])
      pltpu.async_copy(edge_hbm.at[1, pl.ds(eb, CHUNK)], dst_v.at[jm],
                       isem.at[jm])

    def iwait(jm):
      pltpu.make_async_copy(edge_hbm.at[0, pl.ds(0, CHUNK)], src_v.at[jm],
                            isem.at[jm]).wait()
      pltpu.make_async_copy(edge_hbm.at[1, pl.ds(0, CHUNK)], dst_v.at[jm],
                            isem.at[jm]).wait()

    def gstart(jm):
      pltpu.async_copy(table_hbm.at[src_v.at[jm]], rows_v.at[jm],
                       gsem.at[jm])

    def gwait(jm):
      pltpu.make_async_copy(table_hbm.at[pl.ds(0, CHUNK)], rows_v.at[jm],
                            gsem.at[jm]).wait()

    def sstart(jm):
      pltpu.async_copy(rows_v.at[jm], acc_sh.at[dst_v.at[jm]], ssem.at[jm],
                       add=True)

    def swait(jm):
      pltpu.make_async_copy(rows_v.at[jm], acc_sh.at[pl.ds(0, CHUNK)],
                            ssem.at[jm]).wait()

    istart(0, 0)
    istart(1, 1)
    iwait(0)
    gstart(0)

    def step(k, carry):
      km = k % NSLOT
      k1m = (k + 1) % NSLOT
      k2m = (k + 2) % NSLOT

      @pl.when(k >= SWAIT_LAG)
      def _():
        swait((k - SWAIT_LAG) % NSLOT)

      @pl.when(k + 2 < NCHUNK)
      def _():
        istart(k + 2, k2m)

      @pl.when(k + 1 < NCHUNK)
      def _():
        iwait(k1m)
        gstart(k1m)

      gwait(km)
      sstart(km)
      return carry

    lax.fori_loop(0, NCHUNK, step, 0)
    for j in range(max(0, NCHUNK - SWAIT_LAG), NCHUNK):
      swait(j % NSLOT)
    plsc.subcore_barrier()

    pltpu.sync_copy(acc_sh.at[pl.ds(ibase, INIT_STEP)],
                    out_hbm.at[c, pl.ds(ibase, INIT_STEP)])

    @pl.when(s == 15)
    def _():
      pltpu.sync_copy(acc_sh.at[pl.ds(16 * INIT_STEP, INIT_TAIL)],
                      out_hbm.at[c, pl.ds(16 * INIT_STEP, INIT_TAIL)])

  return sc_agg


def _tc1_body(parts_ref, w1_ref, b1_ref, w2_ref, b2_ref, w3_ref,
              g1_ref, bb1_ref, m1_ref, v1_ref, ptab_ref):
  h0 = parts_ref[0] + parts_ref[1]
  h1 = jnp.maximum(jnp.dot(h0, w1_ref[...],
                           preferred_element_type=jnp.float32) + b1_ref[...],
                   0.0)
  h = jnp.dot(h1, w2_ref[...], preferred_element_type=jnp.float32) + b2_ref[...]
  scale = g1_ref[...] * lax.rsqrt(v1_ref[...] + 1e-5)
  h = (h - m1_ref[...]) * scale + bb1_ref[...]
  h = jnp.maximum(h, 0.0)
  ptab_ref[...] = jnp.dot(h, w3_ref[...], preferred_element_type=jnp.float32)


def _tc2_body(parts_ref, batch_ref, b3_ref, w4_ref, b4_ref, g2_ref, bb2_ref,
              m2_ref, v2_ref, out_ref, sums_acc, cnts_acc):
  i = pl.program_id(0)

  @pl.when(i == 0)
  def _():
    sums_acc[...] = jnp.zeros_like(sums_acc)
    cnts_acc[...] = jnp.zeros_like(cnts_acc)

  h2p = parts_ref[0] + parts_ref[1] + b3_ref[...]
  r = jnp.maximum(h2p, 0.0)
  h2 = jnp.dot(r, w4_ref[...], preferred_element_type=jnp.float32) + b4_ref[...]
  scale = g2_ref[...] * lax.rsqrt(v2_ref[...] + 1e-5)
  h2 = (h2 - m2_ref[...]) * scale + bb2_ref[...]
  h2 = jnp.maximum(h2, 0.0)

  seg = batch_ref[0]
  gids = lax.broadcasted_iota(jnp.int32, (N_GRAPHS, TC_BLOCK), 0)
  onehot = (gids == seg).astype(jnp.float32)
  sums_acc[...] += jnp.dot(onehot, h2, preferred_element_type=jnp.float32)
  cnts_acc[...] += jnp.sum(onehot, axis=1, keepdims=True)

  @pl.when(i == TC_GRID - 1)
  def _():
    out_ref[...] = sums_acc[...] / jnp.maximum(cnts_acc[...], 1.0)


def kernel(x, edge_index, batch, W1, b1, W2, b2, W3, b3, W4, b4,
           bn1_g, bn1_b, bn1_m, bn1_v, bn2_g, bn2_b, bn2_m, bn2_v):
  zeros16 = jnp.zeros((ZROWS, 16), jnp.float32)
  zeros8 = jnp.zeros((ZROWS, 8), jnp.float32)

  parts1 = _sc_agg(16)(edge_index, x, zeros16)

  row = lambda a: a.reshape(1, -1)
  full = lambda shape: pl.BlockSpec(shape, lambda i: (0,) * len(shape))

  ptab = pl.pallas_call(
      _tc1_body,
      grid=(TC_GRID,),
      in_specs=[
          pl.BlockSpec((2, TC_BLOCK, 16), lambda i: (0, i, 0)),
          full((16, 64)), full((1, 64)), full((64, 64)), full((1, 64)),
          full((64, 8)),
          full((1, 64)), full((1, 64)), full((1, 64)), full((1, 64)),
      ],
      out_specs=pl.BlockSpec((TC_BLOCK, 8), lambda i: (i, 0)),
      out_shape=jax.ShapeDtypeStruct((N_NODES, 8), jnp.float32),
  )(parts1, W1, row(b1), W2, row(b2), W3,
    row(bn1_g), row(bn1_b), row(bn1_m), row(bn1_v))

  parts2 = _sc_agg(8)(edge_index, ptab, zeros8)

  batch3d = batch.astype(jnp.int32).reshape(TC_GRID, 1, TC_BLOCK)
  out = pl.pallas_call(
      _tc2_body,
      grid=(TC_GRID,),
      in_specs=[
          pl.BlockSpec((2, TC_BLOCK, 8), lambda i: (0, i, 0)),
          pl.BlockSpec((1, 1, TC_BLOCK), lambda i: (i, 0, 0)),
          full((1, 8)), full((8, 8)), full((1, 8)),
          full((1, 8)), full((1, 8)), full((1, 8)), full((1, 8)),
      ],
      out_specs=pl.BlockSpec((N_GRAPHS, 8), lambda i: (0, 0)),
      out_shape=jax.ShapeDtypeStruct((N_GRAPHS, 8), jnp.float32),
      scratch_shapes=[
          pltpu.VMEM((N_GRAPHS, 8), jnp.float32),
          pltpu.VMEM((N_GRAPHS, 1), jnp.float32),
      ],
  )(parts2, batch3d, row(b3), W4, row(b4),
    row(bn2_g), row(bn2_b), row(bn2_m), row(bn2_v))

  return out[:, None, :]
